# baseline (speedup 1.0000x reference)
import functools

import jax
import jax.numpy as jnp
from jax import lax
from jax.experimental import pallas as pl
from jax.experimental.pallas import tpu as pltpu
from jax.experimental.pallas import tpu_sc as plsc

_H = 1024
_F = 2048
_E = 8
_T = 32
_BF = 1024
_NF = _F // _BF

_NC = 2
_NS = 16
_NW = _NC * _NS
_L = 16

_mesh = plsc.VectorSubcoreMesh(core_axis_name="c", subcore_axis_name="s")


@functools.partial(
    pl.kernel,
    out_type=jax.ShapeDtypeStruct((_T, _L), jnp.float32),
    mesh=_mesh,
    compiler_params=pltpu.CompilerParams(needs_layout_passes=False),
    scratch_types=[
        pltpu.VMEM((1, _H), jnp.float32),
        pltpu.VMEM((_E, _H), jnp.float32),
        pltpu.VMEM((1, _L), jnp.float32),
    ],
)
def _sc_router(x_hbm, gw_hbm, rw_hbm, xrow_v, gw_v, rw_v):
    wid = lax.axis_index("s") * _NC + lax.axis_index("c")
    pltpu.sync_copy(x_hbm.at[pl.ds(wid, 1)], xrow_v)
    pltpu.sync_copy(gw_hbm, gw_v)
    iota = lax.iota(jnp.int32, _L)
    logits = jnp.full((_L,), -jnp.inf, jnp.float32)
    for e in range(_E):
        def body(j, acc, e=e):
            return acc + xrow_v[0, pl.ds(j * _L, _L)] * gw_v[e, pl.ds(j * _L, _L)]
        acc = lax.fori_loop(0, _H // _L, body, jnp.zeros((_L,), jnp.float32))
        logits = jnp.where(iota == e, jnp.sum(acc), logits)
    m0 = jnp.max(logits)
    i0 = jnp.min(jnp.where(logits == m0, iota, _L))
    hot0 = iota == i0
    masked = jnp.where(hot0, -jnp.inf, logits)
    m1 = jnp.max(masked)
    i1 = jnp.min(jnp.where(masked == m1, iota, _L))
    hot1 = iota == i1
    r = jnp.max(jnp.exp(jnp.maximum(masked - m0, -80.0)))
    denom = 1.0 + r
    rw_v[0, :] = (jnp.where(hot0, 1.0, 0.0) + jnp.where(hot1, r, 0.0)) / denom
    pltpu.sync_copy(rw_v, rw_hbm.at[pl.ds(wid, 1)])


@functools.partial(
    pl.kernel,
    out_type=jax.ShapeDtypeStruct((_T, _H), jnp.float32),
    mesh=_mesh,
    compiler_params=pltpu.CompilerParams(needs_layout_passes=False),
    scratch_types=[
        pltpu.VMEM((1, _L), jnp.float32),
        pltpu.VMEM((_E, _H), jnp.float32),
        pltpu.VMEM((1, _H), jnp.float32),
        pltpu.SemaphoreType.DMA,
    ],
)
def _sc_combine(oute_hbm, rw_hbm, out_hbm, rw_v, rows_v, orow_v, sem):
    wid = lax.axis_index("s") * _NC + lax.axis_index("c")
    pltpu.sync_copy(rw_hbm.at[pl.ds(wid, 1)], rw_v)
    copies = [
        pltpu.async_copy(oute_hbm.at[e, pl.ds(wid, 1)],
                         rows_v.at[pl.ds(e, 1)], sem)
        for e in range(_E)
    ]
    for c in copies:
        c.wait()
    rwvec = rw_v[0, :]
    ws = [rwvec[e] for e in range(_E)]
    for j in range(_H // _L):
        acc = jnp.zeros((_L,), jnp.float32)
        for e in range(_E):
            acc = acc + ws[e] * rows_v[e, pl.ds(j * _L, _L)]
        orow_v[0, pl.ds(j * _L, _L)] = acc
    pltpu.sync_copy(orow_v, out_hbm.at[pl.ds(wid, 1)])


def _moe_body(x_ref, w13_ref, w2_ref, out_ref):
    f = pl.program_id(1)
    x = x_ref[...]

    def mm(a, b):
        return jax.lax.dot_general(a, b, (((1,), (1,)), ((), ())),
                                   preferred_element_type=jnp.float32)

    h1 = mm(x, w13_ref[0, 0])
    h3 = mm(x, w13_ref[0, 1])
    act = h1 * jax.nn.sigmoid(h1) * h3
    contrib = mm(act, w2_ref[0])

    @pl.when(f == 0)
    def _init():
        out_ref[0] = contrib

    @pl.when(f != 0)
    def _acc():
        out_ref[0] += contrib


def _tc_experts(x, w13r, w2):
    return pl.pallas_call(
        _moe_body,
        grid=(_E, _NF),
        in_specs=[
            pl.BlockSpec((_T, _H), lambda e, f: (0, 0)),
            pl.BlockSpec((1, 2, _BF, _H), lambda e, f: (e, 0, f, 0)),
            pl.BlockSpec((1, _H, _BF), lambda e, f: (e, 0, f)),
        ],
        out_specs=pl.BlockSpec((1, _T, _H), lambda e, f: (e, 0, 0)),
        out_shape=jax.ShapeDtypeStruct((_E, _T, _H), jnp.float32),
        compiler_params=pltpu.CompilerParams(
            dimension_semantics=("arbitrary", "arbitrary"),
        ),
    )(x, w13r, w2)


@jax.jit
def kernel(x, gate_w, w13, w2):
    w13r = w13.reshape(_E, 2, _F, _H)
    rw = _sc_router(x, gate_w)
    out_e = _tc_experts(x, w13r, w2)
    return _sc_combine(out_e, rw)
